# baseline (speedup 1.0000x reference)
import jax
import jax.numpy as jnp
from jax.experimental import pallas as pl
from jax.experimental.pallas import tpu as pltpu

_N, _D = 16384, 1000
_R = 256
_G = _N // _R
_NBUF = 4


def _outer(tgt_hbm, inp_hbm, out_ref):
    out_ref[...] = jnp.zeros((1, 1), jnp.float32)

    def _step(tgt_ref, inp_ref):
        x = inp_ref[...]
        t = tgt_ref[0, 0, :]
        col = jax.lax.broadcasted_iota(jnp.int32, (_R, _D), 1)
        is_t = col == t[:, None]
        v_y = jnp.sum(jnp.where(is_t, x, 0.0), axis=1)
        m_neq = jnp.max(jnp.where(is_t, -jnp.inf, x), axis=1)
        loss = jnp.clip(1.0 + (m_neq - v_y), 0.0, 1.0)
        out_ref[...] += jnp.sum(loss).reshape(1, 1)

    pltpu.emit_pipeline(
        _step,
        grid=(_G,),
        in_specs=[
            pl.BlockSpec((1, 1, _R), lambda i: (i, 0, 0)),
            pl.BlockSpec((_R, _D), lambda i: (i, 0),
                         pipeline_mode=pl.Buffered(buffer_count=_NBUF)),
        ],
    )(tgt_hbm, inp_hbm)


def kernel(inp, tgt):
    tgt3 = tgt.astype(jnp.int32).reshape(_G, 1, _R)
    out = pl.pallas_call(
        _outer,
        in_specs=[
            pl.BlockSpec(memory_space=pltpu.HBM),
            pl.BlockSpec(memory_space=pltpu.HBM),
        ],
        out_specs=pl.BlockSpec(memory_space=pltpu.VMEM),
        out_shape=jax.ShapeDtypeStruct((1, 1), jnp.float32),
    )(tgt3, inp)
    return (out[0] / _N).reshape(1)


# speedup vs baseline: 1.0455x; 1.0455x over previous
import jax
import jax.numpy as jnp
from jax import lax
from jax.experimental import pallas as pl
from jax.experimental.pallas import tpu as pltpu

_N, _D = 16384, 1000
_R = 256
_G = _N // _R
_NB = 8


def _block_loss_sum(x, t):
    col = jax.lax.broadcasted_iota(jnp.int32, (_R, _D), 1)
    is_t = col == t[:, None]
    v_y = jnp.sum(jnp.where(is_t, x, 0.0), axis=1)
    m_neq = jnp.max(jnp.where(is_t, -jnp.inf, x), axis=1)
    loss = jnp.clip(1.0 + (m_neq - v_y), 0.0, 1.0)
    return jnp.sum(loss)


def _outer(tgt_hbm, inp_hbm, out_ref, bufs, sems, tbuf, tsem):
    pltpu.make_async_copy(tgt_hbm, tbuf, tsem).start()
    for b in range(_NB):
        pltpu.make_async_copy(
            inp_hbm.at[pl.ds(b * _R, _R), :], bufs.at[b], sems.at[b]
        ).start()
    pltpu.make_async_copy(tgt_hbm, tbuf, tsem).wait()

    def body(j, acc):
        slot = lax.rem(j, _NB)
        pltpu.make_async_copy(
            inp_hbm.at[pl.ds(j * _R, _R), :], bufs.at[slot], sems.at[slot]
        ).wait()
        x = bufs[slot]
        t = tbuf[j]
        acc = acc + _block_loss_sum(x, t)

        @pl.when(j + _NB < _G)
        def _():
            pltpu.make_async_copy(
                inp_hbm.at[pl.ds((j + _NB) * _R, _R), :],
                bufs.at[slot],
                sems.at[slot],
            ).start()

        return acc

    acc = lax.fori_loop(0, _G, body, jnp.float32(0.0))
    out_ref[...] = acc.reshape(1, 1)


def kernel(inp, tgt):
    tgt2 = tgt.astype(jnp.int32).reshape(_G, _R)
    out = pl.pallas_call(
        _outer,
        in_specs=[
            pl.BlockSpec(memory_space=pltpu.HBM),
            pl.BlockSpec(memory_space=pltpu.HBM),
        ],
        out_specs=pl.BlockSpec(memory_space=pltpu.VMEM),
        out_shape=jax.ShapeDtypeStruct((1, 1), jnp.float32),
        scratch_shapes=[
            pltpu.VMEM((_NB, _R, _D), jnp.float32),
            pltpu.SemaphoreType.DMA((_NB,)),
            pltpu.VMEM((_G, _R), jnp.int32),
            pltpu.SemaphoreType.DMA,
        ],
    )(tgt2, inp)
    return (out[0] / _N).reshape(1)


# speedup vs baseline: 3.5929x; 3.4364x over previous
import jax
import jax.numpy as jnp
from jax import lax
from jax.experimental import pallas as pl
from jax.experimental.pallas import tpu as pltpu

_N, _D = 16384, 1000
_C = 1024
_G = _N // _C
_NB = 6


def _block_loss_sum(x, t):
    row = jax.lax.broadcasted_iota(jnp.int32, (_D, _C), 0)
    is_t = row == t[None, :]
    v_y = jnp.sum(jnp.where(is_t, x, 0.0), axis=0)
    m_neq = jnp.max(jnp.where(is_t, -jnp.inf, x), axis=0)
    loss = jnp.clip(1.0 + (m_neq - v_y), 0.0, 1.0)
    return jnp.sum(loss)


def _outer(tgt_hbm, xt_hbm, out_ref, bufs, sems, tbuf, tsem):
    pltpu.make_async_copy(tgt_hbm, tbuf, tsem).start()
    for b in range(_NB):
        pltpu.make_async_copy(
            xt_hbm.at[:, pl.ds(b * _C, _C)], bufs.at[b], sems.at[b]
        ).start()
    pltpu.make_async_copy(tgt_hbm, tbuf, tsem).wait()

    def body(j, acc):
        slot = lax.rem(j, _NB)
        pltpu.make_async_copy(
            xt_hbm.at[:, pl.ds(j * _C, _C)], bufs.at[slot], sems.at[slot]
        ).wait()
        x = bufs[slot]
        t = tbuf[j]
        acc = acc + _block_loss_sum(x, t)

        @pl.when(j + _NB < _G)
        def _():
            pltpu.make_async_copy(
                xt_hbm.at[:, pl.ds((j + _NB) * _C, _C)],
                bufs.at[slot],
                sems.at[slot],
            ).start()

        return acc

    acc = lax.fori_loop(0, _G, body, jnp.float32(0.0))
    out_ref[...] = acc.reshape(1, 1)


def kernel(inp, tgt):
    xt = inp.T
    tgt2 = tgt.astype(jnp.int32).reshape(_G, _C)
    out = pl.pallas_call(
        _outer,
        in_specs=[
            pl.BlockSpec(memory_space=pltpu.HBM),
            pl.BlockSpec(memory_space=pltpu.HBM),
        ],
        out_specs=pl.BlockSpec(memory_space=pltpu.VMEM),
        out_shape=jax.ShapeDtypeStruct((1, 1), jnp.float32),
        scratch_shapes=[
            pltpu.VMEM((_NB, _D, _C), jnp.float32),
            pltpu.SemaphoreType.DMA((_NB,)),
            pltpu.VMEM((_G, _C), jnp.int32),
            pltpu.SemaphoreType.DMA,
        ],
    )(tgt2, xt)
    return (out[0] / _N).reshape(1)


# speedup vs baseline: 3.6210x; 1.0078x over previous
import jax
import jax.numpy as jnp
from jax import lax
from jax.experimental import pallas as pl
from jax.experimental.pallas import tpu as pltpu

_N, _D = 16384, 1000
_C = 2048
_G = _N // _C
_NB = 4


def _block_loss_sum(x, t):
    row = jax.lax.broadcasted_iota(jnp.int32, (_D, _C), 0)
    is_t = row == t[None, :]
    v_y = jnp.sum(jnp.where(is_t, x, 0.0), axis=0)
    m_neq = jnp.max(jnp.where(is_t, -jnp.inf, x), axis=0)
    loss = jnp.clip(1.0 + (m_neq - v_y), 0.0, 1.0)
    return jnp.sum(loss)


def _outer(tgt_hbm, xt_hbm, out_ref, bufs, sems, tbuf, tsem):
    pltpu.make_async_copy(tgt_hbm, tbuf, tsem).start()
    for b in range(_NB):
        pltpu.make_async_copy(
            xt_hbm.at[:, pl.ds(b * _C, _C)], bufs.at[b], sems.at[b]
        ).start()
    pltpu.make_async_copy(tgt_hbm, tbuf, tsem).wait()

    def body(j, acc):
        slot = lax.rem(j, _NB)
        pltpu.make_async_copy(
            xt_hbm.at[:, pl.ds(j * _C, _C)], bufs.at[slot], sems.at[slot]
        ).wait()
        x = bufs[slot]
        t = tbuf[j]
        acc = acc + _block_loss_sum(x, t)

        @pl.when(j + _NB < _G)
        def _():
            pltpu.make_async_copy(
                xt_hbm.at[:, pl.ds((j + _NB) * _C, _C)],
                bufs.at[slot],
                sems.at[slot],
            ).start()

        return acc

    acc = lax.fori_loop(0, _G, body, jnp.float32(0.0))
    out_ref[...] = acc.reshape(1, 1)


def kernel(inp, tgt):
    xt = inp.T
    tgt2 = tgt.astype(jnp.int32).reshape(_G, _C)
    out = pl.pallas_call(
        _outer,
        in_specs=[
            pl.BlockSpec(memory_space=pltpu.HBM),
            pl.BlockSpec(memory_space=pltpu.HBM),
        ],
        out_specs=pl.BlockSpec(memory_space=pltpu.VMEM),
        out_shape=jax.ShapeDtypeStruct((1, 1), jnp.float32),
        scratch_shapes=[
            pltpu.VMEM((_NB, _D, _C), jnp.float32),
            pltpu.SemaphoreType.DMA((_NB,)),
            pltpu.VMEM((_G, _C), jnp.int32),
            pltpu.SemaphoreType.DMA,
        ],
    )(tgt2, xt)
    return (out[0] / _N).reshape(1)
